# baseline (speedup 1.0000x reference)
import jax
import jax.numpy as jnp
from jax.experimental import pallas as pl
from jax.experimental.pallas import tpu as pltpu

_B, _A, _C = 32, 8400, 80
_EPS = 1e-10
_ALPHA = 0.75
_NB = 4
_STEPS = _B // _NB


def _loss_kernel(ps_ref, ts_ref, pd_ref, tb_ref, lab_ref, a4_ref, out_ref):
    step = pl.program_id(0)

    ps = ps_ref[...]
    ts = ts_ref[...]
    a4 = a4_ref[...]

    iota3 = jax.lax.broadcasted_iota(jnp.int32, (_NB, _C, _A), 1)
    lab3 = lab_ref[:, 0:1, :].astype(jnp.int32)
    oh = iota3 == lab3
    w = jnp.where(oh, ts, _ALPHA * ps * ps)
    logp = jnp.log(ps)
    log1m = jnp.log(1.0 - ps)
    inner = ts * (logp - log1m) + log1m
    m = (inner * w).reshape(_NB * _C, _A)
    iota_c = jax.lax.broadcasted_iota(jnp.int32, (1, _NB * _C), 1)
    ones_c = (iota_c >= 0).astype(jnp.float32)
    row_cls = jax.lax.dot_general(ones_c, m, (((1,), (0,)), ((), ())),
                                  preferred_element_type=jnp.float32)

    row_iota = jax.lax.broadcasted_iota(jnp.int32, (4, 1), 0)
    sign = jnp.where(row_iota < 2, -1.0, 1.0).astype(jnp.float32)
    row_iou = None
    row_tss = None
    for i in range(_NB):
        bw = jnp.sum(ts[i], axis=0, keepdims=True)
        pb = a4[0:4] + sign * pd_ref[i]
        tbt = tb_ref[i]
        fgr = (lab_ref[i, 0:1, :] != float(_C)).astype(jnp.float32)
        b1x1, b1y1, b1x2, b1y2 = pb[0:1], pb[1:2], pb[2:3], pb[3:4]
        b2x1, b2y1, b2x2, b2y2 = tbt[0:1], tbt[1:2], tbt[2:3], tbt[3:4]
        iw = jnp.clip(jnp.minimum(b1x2, b2x2) - jnp.maximum(b1x1, b2x1), 0.0)
        ih = jnp.clip(jnp.minimum(b1y2, b2y2) - jnp.maximum(b1y1, b2y1), 0.0)
        inter = iw * ih
        w1 = b1x2 - b1x1
        h1 = b1y2 - b1y1
        w2 = b2x2 - b2x1
        h2 = b2y2 - b2y1
        union = w1 * h1 + w2 * h2 - inter + _EPS
        iou = inter / union
        cw = jnp.maximum(b1x2, b2x2) - jnp.minimum(b1x1, b2x1)
        ch = jnp.maximum(b1y2, b2y2) - jnp.minimum(b1y1, b2y1)
        c_area = cw * ch + _EPS
        giou = iou - (c_area - union) / c_area
        contrib = (1.0 - giou) * fgr * bw
        row_iou = contrib if row_iou is None else row_iou + contrib
        row_tss = bw if row_tss is None else row_tss + bw

    @pl.when(step == 0)
    def _init():
        out_ref[...] = jnp.zeros((8, _A), jnp.float32)

    out_ref[0:1] += row_cls
    out_ref[1:2] += row_iou
    out_ref[2:3] += row_tss


def kernel(pred_scores, pred_distri, anchor_points_s, target_bboxes,
           target_scores, target_labels, fg_mask):
    psT = pred_scores.transpose(0, 2, 1)
    tsT = target_scores.transpose(0, 2, 1)
    pdt = pred_distri.transpose(0, 2, 1)
    tbt = target_bboxes.transpose(0, 2, 1)
    lab = jnp.where(fg_mask, target_labels, _C).astype(jnp.float32)
    lab4 = jnp.concatenate(
        [lab[:, None, :], jnp.zeros((_B, 3, _A), jnp.float32)], axis=1)
    apt = anchor_points_s.T
    a4 = jnp.concatenate([apt, apt, jnp.zeros((4, _A), jnp.float32)], axis=0)

    rows = pl.pallas_call(
        _loss_kernel,
        grid=(_STEPS,),
        in_specs=[
            pl.BlockSpec((_NB, _C, _A), lambda b: (b, 0, 0)),
            pl.BlockSpec((_NB, _C, _A), lambda b: (b, 0, 0)),
            pl.BlockSpec((_NB, 4, _A), lambda b: (b, 0, 0)),
            pl.BlockSpec((_NB, 4, _A), lambda b: (b, 0, 0)),
            pl.BlockSpec((_NB, 4, _A), lambda b: (b, 0, 0)),
            pl.BlockSpec((8, _A), lambda b: (0, 0)),
        ],
        out_specs=pl.BlockSpec((8, _A), lambda b: (0, 0)),
        out_shape=jax.ShapeDtypeStruct((8, _A), jnp.float32),
    )(psT, tsT, pdt, tbt, lab4, a4)

    s_cls = -jnp.sum(rows[0])
    s_iou = jnp.sum(rows[1])
    s_tss = jnp.sum(rows[2])
    return (s_cls + 2.5 * s_iou) / s_tss


# speedup vs baseline: 1.0243x; 1.0243x over previous
import jax
import jax.numpy as jnp
from jax.experimental import pallas as pl
from jax.experimental.pallas import tpu as pltpu

_B, _A, _C = 32, 8400, 80
_EPS = 1e-10
_ALPHA = 0.75
_NB = 2
_STEPS = _B // _NB


def _loss_kernel(ps_ref, ts_ref, pd_ref, tb_ref, lab_ref, a4_ref, out_ref):
    step = pl.program_id(0)

    ps = ps_ref[...]
    ts = ts_ref[...]
    a4 = a4_ref[...]

    iota3 = jax.lax.broadcasted_iota(jnp.int32, (_NB, _C, _A), 1)
    lab3 = lab_ref[:, 0:1, :].astype(jnp.int32)
    oh = iota3 == lab3
    w = jnp.where(oh, ts, _ALPHA * ps * ps)
    logp = jnp.log(ps)
    log1m = jnp.log(1.0 - ps)
    inner = ts * (logp - log1m) + log1m
    m = (inner * w).reshape(_NB * _C, _A)
    iota_c = jax.lax.broadcasted_iota(jnp.int32, (1, _NB * _C), 1)
    ones_c = (iota_c >= 0).astype(jnp.float32)
    row_cls = jax.lax.dot_general(ones_c, m, (((1,), (0,)), ((), ())),
                                  preferred_element_type=jnp.float32)

    row_iota = jax.lax.broadcasted_iota(jnp.int32, (4, 1), 0)
    sign = jnp.where(row_iota < 2, -1.0, 1.0).astype(jnp.float32)
    row_iou = None
    row_tss = None
    for i in range(_NB):
        bw = jnp.sum(ts[i], axis=0, keepdims=True)
        pb = a4[0:4] + sign * pd_ref[i]
        tbt = tb_ref[i]
        fgr = (lab_ref[i, 0:1, :] != float(_C)).astype(jnp.float32)
        b1x1, b1y1, b1x2, b1y2 = pb[0:1], pb[1:2], pb[2:3], pb[3:4]
        b2x1, b2y1, b2x2, b2y2 = tbt[0:1], tbt[1:2], tbt[2:3], tbt[3:4]
        iw = jnp.clip(jnp.minimum(b1x2, b2x2) - jnp.maximum(b1x1, b2x1), 0.0)
        ih = jnp.clip(jnp.minimum(b1y2, b2y2) - jnp.maximum(b1y1, b2y1), 0.0)
        inter = iw * ih
        w1 = b1x2 - b1x1
        h1 = b1y2 - b1y1
        w2 = b2x2 - b2x1
        h2 = b2y2 - b2y1
        union = w1 * h1 + w2 * h2 - inter + _EPS
        iou = inter / union
        cw = jnp.maximum(b1x2, b2x2) - jnp.minimum(b1x1, b2x1)
        ch = jnp.maximum(b1y2, b2y2) - jnp.minimum(b1y1, b2y1)
        c_area = cw * ch + _EPS
        giou = iou - (c_area - union) / c_area
        contrib = (1.0 - giou) * fgr * bw
        row_iou = contrib if row_iou is None else row_iou + contrib
        row_tss = bw if row_tss is None else row_tss + bw

    @pl.when(step == 0)
    def _init():
        out_ref[...] = jnp.zeros((8, _A), jnp.float32)

    out_ref[0:1] += row_cls
    out_ref[1:2] += row_iou
    out_ref[2:3] += row_tss


def kernel(pred_scores, pred_distri, anchor_points_s, target_bboxes,
           target_scores, target_labels, fg_mask):
    psT = pred_scores.transpose(0, 2, 1)
    tsT = target_scores.transpose(0, 2, 1)
    pdt = pred_distri.transpose(0, 2, 1)
    tbt = target_bboxes.transpose(0, 2, 1)
    lab = jnp.where(fg_mask, target_labels, _C).astype(jnp.float32)
    lab4 = jnp.concatenate(
        [lab[:, None, :], jnp.zeros((_B, 3, _A), jnp.float32)], axis=1)
    apt = anchor_points_s.T
    a4 = jnp.concatenate([apt, apt, jnp.zeros((4, _A), jnp.float32)], axis=0)

    rows = pl.pallas_call(
        _loss_kernel,
        grid=(_STEPS,),
        in_specs=[
            pl.BlockSpec((_NB, _C, _A), lambda b: (b, 0, 0)),
            pl.BlockSpec((_NB, _C, _A), lambda b: (b, 0, 0)),
            pl.BlockSpec((_NB, 4, _A), lambda b: (b, 0, 0)),
            pl.BlockSpec((_NB, 4, _A), lambda b: (b, 0, 0)),
            pl.BlockSpec((_NB, 4, _A), lambda b: (b, 0, 0)),
            pl.BlockSpec((8, _A), lambda b: (0, 0)),
        ],
        out_specs=pl.BlockSpec((8, _A), lambda b: (0, 0)),
        out_shape=jax.ShapeDtypeStruct((8, _A), jnp.float32),
    )(psT, tsT, pdt, tbt, lab4, a4)

    s_cls = -jnp.sum(rows[0])
    s_iou = jnp.sum(rows[1])
    s_tss = jnp.sum(rows[2])
    return (s_cls + 2.5 * s_iou) / s_tss
